# baseline (speedup 1.0000x reference)
import functools

import jax
import jax.numpy as jnp
from jax import lax
from jax.experimental import pallas as pl
from jax.experimental.pallas import tpu as pltpu
from jax.experimental.pallas import tpu_sc as plsc

_CHUNK = 16


def _interleave_sc(text_hbm, img_hbm, idx_hbm, out_hbm,
                   idx_v, buf, row_scratch, sin0, sin1, sout0, sout1, srow,
                   *, rows_per_w, n_idx, num_cores):
    c = lax.axis_index("c")
    s = lax.axis_index("s")
    wid = s * num_cores + c
    base = wid * rows_per_w
    nch = rows_per_w // _CHUNK

    def chunk_slice(k):
        return pl.ds(base + k * _CHUNK, _CHUNK)

    sins = (sin0, sin1)
    souts = (sout0, sout1)

    pltpu.make_async_copy(text_hbm.at[chunk_slice(0)], buf.at[0], sin0).start()
    pltpu.make_async_copy(text_hbm.at[chunk_slice(1)], buf.at[1], sin1).start()

    pltpu.sync_copy(idx_hbm, idx_v.at[pl.ds(0, n_idx)])

    lo_vec = jnp.full((16,), base, jnp.int32)
    hi_vec = jnp.full((16,), base + rows_per_w, jnp.int32)
    ones = jnp.full((16,), 1, jnp.int32)
    zeros = jnp.full((16,), 0, jnp.int32)

    def scan_body(i, carry):
        st, en = carry
        v = idx_v[pl.ds(i * 16, 16)]
        st = st + jnp.where(v < lo_vec, ones, zeros)
        en = en + jnp.where(v < hi_vec, ones, zeros)
        return st, en

    st_vec, en_vec = lax.fori_loop(0, n_idx // 16, scan_body, (zeros, zeros))
    start = st_vec[0]
    end = en_vec[0]
    for lane in range(1, 16):
        start = start + st_vec[lane]
        end = end + en_vec[lane]

    def process_chunk(k, b, p, refill):
        rlo = base + k * _CHUNK
        pltpu.make_async_copy(text_hbm.at[chunk_slice(k)], buf.at[b],
                              sins[b]).wait()

        v = idx_v[pl.ds(p, 16)]
        hiv = jnp.full((16,), rlo + _CHUNK, jnp.int32)
        inwin = jnp.where(v < hiv, ones, zeros)
        cnt = inwin[0]
        for lane in range(1, 16):
            cnt = cnt + inwin[lane]
        cnt = jnp.minimum(cnt, end - p)

        def row_issue(t, carry):
            r = idx_v[pl.ds(p + t, 16)][0] - rlo
            pltpu.make_async_copy(img_hbm.at[pl.ds(p + t, 1)],
                                  buf.at[b].at[pl.ds(r, 1)], srow).start()
            return carry

        lax.fori_loop(0, cnt, row_issue, jnp.int32(0))

        def drain(_, carry):
            pltpu.make_async_copy(img_hbm.at[pl.ds(0, 1)], row_scratch,
                                  srow).wait()
            return carry

        lax.fori_loop(0, cnt, drain, jnp.int32(0))

        pltpu.make_async_copy(buf.at[b], out_hbm.at[chunk_slice(k)],
                              souts[b]).start()

        if refill:
            pltpu.make_async_copy(buf.at[b], out_hbm.at[chunk_slice(k)],
                                  souts[b]).wait()
            pltpu.make_async_copy(text_hbm.at[chunk_slice(k + 2)], buf.at[b],
                                  sins[b]).start()

        return p + cnt

    def body(i, p):
        p = process_chunk(2 * i, 0, p, True)
        p = process_chunk(2 * i + 1, 1, p, True)
        return p

    p = lax.fori_loop(0, nch // 2 - 1, body, start)
    p = process_chunk(nch - 2, 0, p, False)
    p = process_chunk(nch - 1, 1, p, False)

    pltpu.make_async_copy(buf.at[0], out_hbm.at[chunk_slice(nch - 2)],
                          sout0).wait()
    pltpu.make_async_copy(buf.at[1], out_hbm.at[chunk_slice(nch - 1)],
                          sout1).wait()


def kernel(image_embeddings, text_embeddings, vision_indices):
    b, seq, d = text_embeddings.shape
    n_rows = b * seq
    text = text_embeddings.reshape(n_rows, d)
    img = image_embeddings.reshape(-1, d)
    n_idx = img.shape[0]
    idx = vision_indices.reshape(-1).astype(jnp.int32)

    info = plsc.get_sparse_core_info()
    num_cores, num_subcores = info.num_cores, info.num_subcores
    rows_per_w = n_rows // (num_cores * num_subcores)

    mesh = plsc.VectorSubcoreMesh(core_axis_name="c", subcore_axis_name="s")
    body = functools.partial(
        _interleave_sc,
        rows_per_w=rows_per_w,
        n_idx=n_idx,
        num_cores=num_cores,
    )
    out = pl.kernel(
        body,
        out_type=jax.ShapeDtypeStruct((n_rows, d), text.dtype),
        mesh=mesh,
        scratch_types=[
            pltpu.VMEM((n_idx + 16,), jnp.int32),
            pltpu.VMEM((2, _CHUNK, d), text.dtype),
            pltpu.VMEM((1, d), text.dtype),
            pltpu.SemaphoreType.DMA,
            pltpu.SemaphoreType.DMA,
            pltpu.SemaphoreType.DMA,
            pltpu.SemaphoreType.DMA,
            pltpu.SemaphoreType.DMA,
        ],
    )(text, img, idx)
    return out.reshape(b, seq, d)


# speedup vs baseline: 1.1085x; 1.1085x over previous
import functools

import jax
import jax.numpy as jnp
from jax import lax
from jax.experimental import pallas as pl
from jax.experimental.pallas import tpu as pltpu
from jax.experimental.pallas import tpu_sc as plsc

_CHUNK = 16


def _interleave_sc(text_hbm, img_hbm, idx_hbm, out_hbm,
                   idx_v, buf, row_scratch,
                   sin0, sin1, sin2, sout0, sout1, sout2, srow,
                   *, rows_per_w, n_idx, num_cores):
    c = lax.axis_index("c")
    s = lax.axis_index("s")
    wid = s * num_cores + c
    base = wid * rows_per_w
    nch = rows_per_w // _CHUNK

    def chunk_slice(k):
        return pl.ds(base + k * _CHUNK, _CHUNK)

    sins = (sin0, sin1, sin2)
    souts = (sout0, sout1, sout2)

    pltpu.make_async_copy(text_hbm.at[chunk_slice(0)], buf.at[0], sin0).start()
    pltpu.make_async_copy(text_hbm.at[chunk_slice(1)], buf.at[1], sin1).start()
    pltpu.make_async_copy(text_hbm.at[chunk_slice(2)], buf.at[2], sin2).start()

    pltpu.sync_copy(idx_hbm, idx_v.at[pl.ds(0, n_idx)])

    lo_vec = jnp.full((16,), base, jnp.int32)
    hi_vec = jnp.full((16,), base + rows_per_w, jnp.int32)
    ones = jnp.full((16,), 1, jnp.int32)
    zeros = jnp.full((16,), 0, jnp.int32)

    def scan_body(i, carry):
        st, en = carry
        v = idx_v[pl.ds(i * 16, 16)]
        st = st + jnp.where(v < lo_vec, ones, zeros)
        en = en + jnp.where(v < hi_vec, ones, zeros)
        return st, en

    st_vec, en_vec = lax.fori_loop(0, n_idx // 16, scan_body, (zeros, zeros))
    start = st_vec[0]
    end = en_vec[0]
    for lane in range(1, 16):
        start = start + st_vec[lane]
        end = end + en_vec[lane]

    def process_chunk(k, b, p, refill):
        rlo = base + k * _CHUNK
        pltpu.make_async_copy(text_hbm.at[chunk_slice(k)], buf.at[b],
                              sins[b]).wait()

        v = idx_v[pl.ds(p, 16)]
        hiv = jnp.full((16,), rlo + _CHUNK, jnp.int32)
        inwin = jnp.where(v < hiv, ones, zeros)
        cnt = inwin[0]
        for lane in range(1, 16):
            cnt = cnt + inwin[lane]
        cnt = jnp.minimum(cnt, end - p)

        def row_issue(t, carry):
            r = idx_v[pl.ds(p + t, 16)][0] - rlo
            pltpu.make_async_copy(img_hbm.at[pl.ds(p + t, 1)],
                                  buf.at[b].at[pl.ds(r, 1)], srow).start()
            return carry

        lax.fori_loop(0, cnt, row_issue, jnp.int32(0))

        def drain(_, carry):
            pltpu.make_async_copy(img_hbm.at[pl.ds(0, 1)], row_scratch,
                                  srow).wait()
            return carry

        lax.fori_loop(0, cnt, drain, jnp.int32(0))

        pltpu.make_async_copy(buf.at[b], out_hbm.at[chunk_slice(k)],
                              souts[b]).start()

        if refill:
            bp = (b + 2) % 3
            pltpu.make_async_copy(buf.at[bp], out_hbm.at[chunk_slice(k - 1)],
                                  souts[bp]).wait()
            pltpu.make_async_copy(text_hbm.at[chunk_slice(k + 2)], buf.at[bp],
                                  sins[bp]).start()

        return p + cnt

    p = process_chunk(0, 0, start, False)

    def body(i, p):
        k = 3 * i + 1
        p = process_chunk(k, 1, p, True)
        p = process_chunk(k + 1, 2, p, True)
        p = process_chunk(k + 2, 0, p, True)
        return p

    p = lax.fori_loop(0, (nch - 5) // 3, body, p)
    p = process_chunk(nch - 4, 1, p, True)
    p = process_chunk(nch - 3, 2, p, True)
    p = process_chunk(nch - 2, 0, p, False)
    p = process_chunk(nch - 1, 1, p, False)

    pltpu.make_async_copy(buf.at[2], out_hbm.at[chunk_slice(nch - 3)],
                          sout2).wait()
    pltpu.make_async_copy(buf.at[0], out_hbm.at[chunk_slice(nch - 2)],
                          sout0).wait()
    pltpu.make_async_copy(buf.at[1], out_hbm.at[chunk_slice(nch - 1)],
                          sout1).wait()


def kernel(image_embeddings, text_embeddings, vision_indices):
    b, seq, d = text_embeddings.shape
    n_rows = b * seq
    text = text_embeddings.reshape(n_rows, d)
    img = image_embeddings.reshape(-1, d)
    n_idx = img.shape[0]
    idx = vision_indices.reshape(-1).astype(jnp.int32)

    info = plsc.get_sparse_core_info()
    num_cores, num_subcores = info.num_cores, info.num_subcores
    rows_per_w = n_rows // (num_cores * num_subcores)

    mesh = plsc.VectorSubcoreMesh(core_axis_name="c", subcore_axis_name="s")
    body = functools.partial(
        _interleave_sc,
        rows_per_w=rows_per_w,
        n_idx=n_idx,
        num_cores=num_cores,
    )
    out = pl.kernel(
        body,
        out_type=jax.ShapeDtypeStruct((n_rows, d), text.dtype),
        mesh=mesh,
        scratch_types=[
            pltpu.VMEM((n_idx + 16,), jnp.int32),
            pltpu.VMEM((3, _CHUNK, d), text.dtype),
            pltpu.VMEM((1, d), text.dtype),
            pltpu.SemaphoreType.DMA,
            pltpu.SemaphoreType.DMA,
            pltpu.SemaphoreType.DMA,
            pltpu.SemaphoreType.DMA,
            pltpu.SemaphoreType.DMA,
            pltpu.SemaphoreType.DMA,
            pltpu.SemaphoreType.DMA,
        ],
    )(text, img, idx)
    return out.reshape(b, seq, d)
